# Initial kernel scaffold; baseline (speedup 1.0000x reference)
import jax
import jax.numpy as jnp
from jax.experimental import pallas as pl


def kernel(next_token_logits, next_hidden_states, energies):
    raise NotImplementedError("write your pallas kernel here")


# speedup vs baseline: 1.2902x; 1.2902x over previous
import functools

import jax
import jax.numpy as jnp
from jax.experimental import pallas as pl

_B = 8
_V = 1000000
_VPAD = 1048576
_ROWS = 8192
_RGRID = 64
_NEG = -jnp.inf
_IBIG = 2**30


def _scan_body(x_ref, topv_ref, topi_ref, stats_ref):
    x = x_ref[0]
    x3 = x.reshape(_RGRID, 128, 128)
    rm = jnp.max(x3, axis=-1)
    m = jnp.max(rm)
    s = jnp.sum(jnp.exp(x - m))

    riota = (jax.lax.broadcasted_iota(jnp.int32, (_RGRID, 128), 0) * 128
             + jax.lax.broadcasted_iota(jnp.int32, (_RGRID, 128), 1))
    rows = []
    rmw = rm
    for _ in range(_B):
        v = jnp.max(rmw)
        r = jnp.min(jnp.where(rmw >= v, riota, _IBIG))
        rows.append(r)
        rmw = jnp.where(riota == r, _NEG, rmw)

    lane = jax.lax.broadcasted_iota(jnp.int32, (1, 128), 1)
    cand = jnp.concatenate([x_ref[0, pl.ds(r, 1), :] for r in rows], axis=0)
    lidx = jnp.concatenate([r * 128 + lane for r in rows], axis=0)

    lane8 = jax.lax.broadcasted_iota(jnp.int32, (1, _B), 1)
    vacc = jnp.zeros((1, _B), jnp.float32)
    iacc = jnp.zeros((1, _B), jnp.int32)
    for j in range(_B):
        v = jnp.max(cand)
        li = jnp.min(jnp.where(cand >= v, lidx, _IBIG))
        vacc = jnp.where(lane8 == j, v, vacc)
        iacc = jnp.where(lane8 == j, li, iacc)
        cand = jnp.where(lidx == li, _NEG, cand)

    topv_ref[0] = vacc
    topi_ref[0] = iacc
    st = jnp.where(lane8 == 0, m, jnp.where(lane8 == 1, s, 0.0))
    stats_ref[0] = st


def _merge_body(topv_ref, topi_ref, stats_ref, energy_ref, hid_ref,
                act_ref, logp_ref, eng_ref, tok_ref, hid_out_ref):
    topv = topv_ref[...]
    topi = topi_ref[...]
    stats = stats_ref[...]
    energy = energy_ref[...]

    m = stats[:, 0:1]
    s = stats[:, 1:2]
    cand_logp = (topv - m) - jnp.log(s)
    action = energy - cand_logp

    cidx = (jax.lax.broadcasted_iota(jnp.int32, (_B, _B), 0) * _B
            + jax.lax.broadcasted_iota(jnp.int32, (_B, _B), 1))
    lane8 = jax.lax.broadcasted_iota(jnp.int32, (1, _B), 1)
    row8 = jax.lax.broadcasted_iota(jnp.int32, (_B, _B), 0)
    col8 = jax.lax.broadcasted_iota(jnp.int32, (_B, _B), 1)

    a_acc = jnp.zeros((1, _B), jnp.float32)
    lp_acc = jnp.zeros((1, _B), jnp.float32)
    en_acc = jnp.zeros((1, _B), jnp.float32)
    tk_acc = jnp.zeros((1, _B), jnp.int32)
    onehot = jnp.zeros((_B, _B), jnp.float32)

    aw = action
    for j in range(_B):
        a = jnp.min(aw)
        sel = jnp.min(jnp.where(aw <= a, cidx, _IBIG))
        hit = cidx == sel
        a_acc = jnp.where(lane8 == j, a, a_acc)
        lp_acc = jnp.where(lane8 == j, jnp.sum(jnp.where(hit, cand_logp, 0.0)),
                           lp_acc)
        en_acc = jnp.where(lane8 == j,
                           jnp.sum(jnp.where(hit,
                                             jnp.broadcast_to(energy,
                                                              (_B, _B)),
                                             0.0)), en_acc)
        tk_acc = jnp.where(lane8 == j, jnp.sum(jnp.where(hit, topi, 0)),
                           tk_acc)
        beam = sel // _B
        onehot = onehot + jnp.where((row8 == j) & (col8 == beam), 1.0, 0.0)
        aw = jnp.where(hit, jnp.inf, aw)

    act_ref[...] = a_acc
    logp_ref[...] = lp_acc
    eng_ref[...] = en_acc
    tok_ref[...] = tk_acc
    hid_out_ref[...] = jax.lax.dot(onehot, hid_ref[...],
                                   preferred_element_type=jnp.float32)


@functools.partial(jax.jit, static_argnames=("interpret",))
def _impl(next_token_logits, next_hidden_states, energies, interpret=False):
    xp = jnp.pad(next_token_logits, ((0, 0), (0, _VPAD - _V)),
                 constant_values=-jnp.inf)
    x3 = xp.reshape(_B, _ROWS, 128)
    topv, topi, stats = pl.pallas_call(
        _scan_body,
        grid=(_B,),
        in_specs=[pl.BlockSpec((1, _ROWS, 128), lambda b: (b, 0, 0))],
        out_specs=[
            pl.BlockSpec((1, 1, _B), lambda b: (b, 0, 0)),
            pl.BlockSpec((1, 1, _B), lambda b: (b, 0, 0)),
            pl.BlockSpec((1, 1, _B), lambda b: (b, 0, 0)),
        ],
        out_shape=[
            jax.ShapeDtypeStruct((_B, 1, _B), jnp.float32),
            jax.ShapeDtypeStruct((_B, 1, _B), jnp.int32),
            jax.ShapeDtypeStruct((_B, 1, _B), jnp.float32),
        ],
        interpret=interpret,
    )(x3)

    act, logp, eng, tok, hid = pl.pallas_call(
        _merge_body,
        out_shape=[
            jax.ShapeDtypeStruct((1, _B), jnp.float32),
            jax.ShapeDtypeStruct((1, _B), jnp.float32),
            jax.ShapeDtypeStruct((1, _B), jnp.float32),
            jax.ShapeDtypeStruct((1, _B), jnp.int32),
            jax.ShapeDtypeStruct((_B, 4096), jnp.float32),
        ],
        interpret=interpret,
    )(topv.reshape(_B, _B), topi.reshape(_B, _B), stats.reshape(_B, _B),
      energies.reshape(_B, 1), next_hidden_states)

    return (act.reshape(_B), logp.reshape(_B), eng.reshape(_B),
            tok.reshape(_B), hid)


def kernel(next_token_logits, next_hidden_states, energies):
    return _impl(next_token_logits, next_hidden_states, energies)


# speedup vs baseline: 1.6163x; 1.2527x over previous
import functools

import jax
import jax.numpy as jnp
from jax.experimental import pallas as pl
from jax.experimental.pallas import tpu as pltpu

_B = 8
_V = 1000000
_C = 32768
_NC = 31
_G = 1024
_GPC = _C // _G
_NG = 977
_NEG = -jnp.inf
_IBIG = 2**30


def _scan_body(x_ref, ids_ref, stats_ref, m_scr, s_scr, gm_scr):
    i = pl.program_id(0)
    x = x_ref[...]
    liota = jax.lax.broadcasted_iota(jnp.int32, (_B, _C), 1)
    x = jnp.where(i * _C + liota < _V, x, _NEG)

    @pl.when(i == 0)
    def _init():
        m_scr[...] = jnp.full((_B, 128), _NEG, jnp.float32)
        s_scr[...] = jnp.zeros((_B, 128), jnp.float32)

    gms = [jnp.max(x[:, j * _G:(j + 1) * _G], axis=1, keepdims=True)
           for j in range(_GPC)]
    l1024 = jax.lax.broadcasted_iota(jnp.int32, (_B, _NC * _GPC), 1)
    acc = gm_scr[...]
    for j in range(_GPC):
        acc = jnp.where(l1024 == i * _GPC + j, gms[j], acc)
    gm_scr[...] = acc

    m_c = functools.reduce(jnp.maximum, gms)
    m_old = m_scr[...]
    m_new = jnp.maximum(m_old, m_c)
    s_c = jnp.sum(jnp.exp(x - m_new[:, 0:1]), axis=1, keepdims=True)
    s_scr[...] = s_scr[...] * jnp.exp(m_old - m_new) + s_c
    m_scr[...] = m_new

    @pl.when(i == _NC - 1)
    def _finish():
        lane8 = jax.lax.broadcasted_iota(jnp.int32, (1, _B), 1)
        gmv = jnp.where(l1024 < _NG, gm_scr[...], _NEG)
        ids = jnp.zeros((_B, _B), jnp.int32)
        for j in range(_B):
            v = jnp.max(gmv, axis=1, keepdims=True)
            sel = jnp.min(jnp.where(gmv >= v, l1024, _IBIG),
                          axis=1, keepdims=True)
            ids = jnp.where(lane8 == j, sel, ids)
            gmv = jnp.where(l1024 == sel, _NEG, gmv)
        ids_ref[...] = ids
        l128 = jax.lax.broadcasted_iota(jnp.int32, (_B, 128), 1)
        stats_ref[...] = jnp.where(l128 == 0, m_scr[...],
                                   jnp.where(l128 == 1, s_scr[...], 0.0))


def _merge_body(ids_ref, x_ref, stats_ref, energy_ref, hid_ref,
                act_ref, logp_ref, eng_ref, tok_ref, hid_out_ref,
                cand_scr):
    b = pl.program_id(0)
    j = pl.program_id(1)
    r = b * _B + j
    li = jax.lax.broadcasted_iota(jnp.int32, (1, _G), 1)
    g = ids_ref[r]
    x = jnp.where(g * _G + li < _V, x_ref[0], _NEG)
    cand_scr[pl.ds(r, 1), :] = x

    @pl.when(r == _B * _B - 1)
    def _finish():
        lane8 = jax.lax.broadcasted_iota(jnp.int32, (1, _B), 1)
        topv_rows = []
        topi_rows = []
        for bb in range(_B):
            cb = cand_scr[bb * _B:(bb + 1) * _B, :]
            lidx = jnp.concatenate(
                [ids_ref[bb * _B + t] * _G + li for t in range(_B)], axis=0)
            vrow = jnp.zeros((1, _B), jnp.float32)
            irow = jnp.zeros((1, _B), jnp.int32)
            for j2 in range(_B):
                v = jnp.max(cb)
                sel = jnp.min(jnp.where(cb >= v, lidx, _IBIG))
                vrow = jnp.where(lane8 == j2, v, vrow)
                irow = jnp.where(lane8 == j2, sel, irow)
                cb = jnp.where(lidx == sel, _NEG, cb)
            topv_rows.append(vrow)
            topi_rows.append(irow)
        topv = jnp.concatenate(topv_rows, axis=0)
        topi = jnp.concatenate(topi_rows, axis=0)

        stats = stats_ref[...]
        m = stats[:, 0:1]
        s = stats[:, 1:2]
        energy = energy_ref[...]
        cand_logp = (topv - m) - jnp.log(s)
        action = energy - cand_logp

        cidx = (jax.lax.broadcasted_iota(jnp.int32, (_B, _B), 0) * _B
                + jax.lax.broadcasted_iota(jnp.int32, (_B, _B), 1))
        row8 = jax.lax.broadcasted_iota(jnp.int32, (_B, _B), 0)
        col8 = jax.lax.broadcasted_iota(jnp.int32, (_B, _B), 1)

        a_acc = jnp.zeros((1, _B), jnp.float32)
        lp_acc = jnp.zeros((1, _B), jnp.float32)
        en_acc = jnp.zeros((1, _B), jnp.float32)
        tk_acc = jnp.zeros((1, _B), jnp.int32)
        onehot = jnp.zeros((_B, _B), jnp.float32)
        energy_b = jnp.broadcast_to(energy, (_B, _B))

        aw = action
        for j2 in range(_B):
            a = jnp.min(aw)
            sel = jnp.min(jnp.where(aw <= a, cidx, _IBIG))
            hit = cidx == sel
            a_acc = jnp.where(lane8 == j2, a, a_acc)
            lp_acc = jnp.where(lane8 == j2,
                               jnp.sum(jnp.where(hit, cand_logp, 0.0)), lp_acc)
            en_acc = jnp.where(lane8 == j2,
                               jnp.sum(jnp.where(hit, energy_b, 0.0)), en_acc)
            tk_acc = jnp.where(lane8 == j2, jnp.sum(jnp.where(hit, topi, 0)),
                               tk_acc)
            beam = sel // _B
            onehot = onehot + jnp.where((row8 == j2) & (col8 == beam),
                                        1.0, 0.0)
            aw = jnp.where(hit, jnp.inf, aw)

        act_ref[...] = a_acc
        logp_ref[...] = lp_acc
        eng_ref[...] = en_acc
        tok_ref[...] = tk_acc
        hid_out_ref[...] = jax.lax.dot(onehot, hid_ref[...],
                                       preferred_element_type=jnp.float32)


@functools.partial(jax.jit, static_argnames=("interpret",))
def _impl(next_token_logits, next_hidden_states, energies, interpret=False):
    ids, stats = pl.pallas_call(
        _scan_body,
        grid=(_NC,),
        in_specs=[pl.BlockSpec((_B, _C), lambda i: (0, i))],
        out_specs=[
            pl.BlockSpec((_B, _B), lambda i: (0, 0)),
            pl.BlockSpec((_B, 128), lambda i: (0, 0)),
        ],
        out_shape=[
            jax.ShapeDtypeStruct((_B, _B), jnp.int32),
            jax.ShapeDtypeStruct((_B, 128), jnp.float32),
        ],
        scratch_shapes=[
            pltpu.VMEM((_B, 128), jnp.float32),
            pltpu.VMEM((_B, 128), jnp.float32),
            pltpu.VMEM((_B, _NC * _GPC), jnp.float32),
        ],
        interpret=interpret,
    )(next_token_logits)

    x3 = next_token_logits.reshape(_B, 1, _V)
    grid_spec = pltpu.PrefetchScalarGridSpec(
        num_scalar_prefetch=1,
        grid=(_B, _B),
        in_specs=[
            pl.BlockSpec((1, 1, _G), lambda b, j, ids: (b, 0, ids[b * _B + j])),
            pl.BlockSpec((_B, 128), lambda b, j, ids: (0, 0)),
            pl.BlockSpec((_B, 1), lambda b, j, ids: (0, 0)),
            pl.BlockSpec((_B, 4096), lambda b, j, ids: (0, 0)),
        ],
        out_specs=[
            pl.BlockSpec((1, _B), lambda b, j, ids: (0, 0)),
            pl.BlockSpec((1, _B), lambda b, j, ids: (0, 0)),
            pl.BlockSpec((1, _B), lambda b, j, ids: (0, 0)),
            pl.BlockSpec((1, _B), lambda b, j, ids: (0, 0)),
            pl.BlockSpec((_B, 4096), lambda b, j, ids: (0, 0)),
        ],
        scratch_shapes=[pltpu.VMEM((_B * _B, _G), jnp.float32)],
    )
    act, logp, eng, tok, hid = pl.pallas_call(
        _merge_body,
        grid_spec=grid_spec,
        out_shape=[
            jax.ShapeDtypeStruct((1, _B), jnp.float32),
            jax.ShapeDtypeStruct((1, _B), jnp.float32),
            jax.ShapeDtypeStruct((1, _B), jnp.float32),
            jax.ShapeDtypeStruct((1, _B), jnp.int32),
            jax.ShapeDtypeStruct((_B, 4096), jnp.float32),
        ],
        interpret=interpret,
    )(ids.reshape(_B * _B), x3, stats, energies.reshape(_B, 1),
      next_hidden_states)

    return (act.reshape(_B), logp.reshape(_B), eng.reshape(_B),
            tok.reshape(_B), hid)


def kernel(next_token_logits, next_hidden_states, energies):
    return _impl(next_token_logits, next_hidden_states, energies)


# speedup vs baseline: 2.1049x; 1.3023x over previous
import functools

import jax
import jax.numpy as jnp
from jax.experimental import pallas as pl
from jax.experimental.pallas import tpu as pltpu

_B = 8
_V = 1000000
_C = 32768
_NC = 31
_G = 1024
_GPC = _C // _G
_NG = 977
_NEG = -jnp.inf
_IBIG = 2**30


def _scan_body(x_ref, ids_ref, stats_ref, m_scr, s_scr, gm_scr):
    i = pl.program_id(0)
    x = x_ref[...]
    liota = jax.lax.broadcasted_iota(jnp.int32, (_B, _C), 1)
    x = jnp.where(i * _C + liota < _V, x, _NEG)

    @pl.when(i == 0)
    def _init():
        m_scr[...] = jnp.full((_B, 128), _NEG, jnp.float32)
        s_scr[...] = jnp.zeros((_B, 128), jnp.float32)

    gms = [jnp.max(x[:, j * _G:(j + 1) * _G], axis=1, keepdims=True)
           for j in range(_GPC)]
    l1024 = jax.lax.broadcasted_iota(jnp.int32, (_B, _NC * _GPC), 1)
    acc = gm_scr[...]
    for j in range(_GPC):
        acc = jnp.where(l1024 == i * _GPC + j, gms[j], acc)
    gm_scr[...] = acc

    m_c = functools.reduce(jnp.maximum, gms)
    m_old = m_scr[...]
    m_new = jnp.maximum(m_old, m_c)
    s_c = jnp.sum(jnp.exp(x - m_new[:, 0:1]), axis=1, keepdims=True)
    s_scr[...] = s_scr[...] * jnp.exp(m_old - m_new) + s_c
    m_scr[...] = m_new

    @pl.when(i == _NC - 1)
    def _finish():
        lane8 = jax.lax.broadcasted_iota(jnp.int32, (1, _B), 1)
        gmv = jnp.where(l1024 < _NG, gm_scr[...], _NEG)
        ids = jnp.zeros((_B, _B), jnp.int32)
        for j in range(_B):
            v = jnp.max(gmv, axis=1, keepdims=True)
            sel = jnp.min(jnp.where(gmv >= v, l1024, _IBIG),
                          axis=1, keepdims=True)
            ids = jnp.where(lane8 == j, sel, ids)
            gmv = jnp.where(l1024 == sel, _NEG, gmv)
        ids_ref[...] = ids
        l128 = jax.lax.broadcasted_iota(jnp.int32, (_B, 128), 1)
        stats_ref[...] = jnp.where(l128 == 0, m_scr[...],
                                   jnp.where(l128 == 1, s_scr[...], 0.0))


def _merge_body(ids_ref, x_ref, stats_ref, energy_ref, hid_ref,
                act_ref, logp_ref, eng_ref, tok_ref, hid_out_ref,
                cand_scr):
    b = pl.program_id(0)
    j = pl.program_id(1)
    r = b * _B + j
    li = jax.lax.broadcasted_iota(jnp.int32, (1, _G), 1)
    g = ids_ref[r]
    xrow = x_ref[pl.ds(b, 1), :]
    x = jnp.where(g * _G + li < _V, xrow, _NEG)
    cand_scr[pl.ds(r, 1), :] = x

    @pl.when(r == _B * _B - 1)
    def _finish():
        lane8 = jax.lax.broadcasted_iota(jnp.int32, (1, _B), 1)
        topv_rows = []
        topi_rows = []
        for bb in range(_B):
            cb = cand_scr[bb * _B:(bb + 1) * _B, :]
            lidx = jnp.concatenate(
                [ids_ref[bb * _B + t] * _G + li for t in range(_B)], axis=0)
            vrow = jnp.zeros((1, _B), jnp.float32)
            irow = jnp.zeros((1, _B), jnp.int32)
            for j2 in range(_B):
                v = jnp.max(cb)
                sel = jnp.min(jnp.where(cb >= v, lidx, _IBIG))
                vrow = jnp.where(lane8 == j2, v, vrow)
                irow = jnp.where(lane8 == j2, sel, irow)
                cb = jnp.where(lidx == sel, _NEG, cb)
            topv_rows.append(vrow)
            topi_rows.append(irow)
        topv = jnp.concatenate(topv_rows, axis=0)
        topi = jnp.concatenate(topi_rows, axis=0)

        stats = stats_ref[...]
        m = stats[:, 0:1]
        s = stats[:, 1:2]
        energy = energy_ref[...]
        cand_logp = (topv - m) - jnp.log(s)
        action = energy - cand_logp

        cidx = (jax.lax.broadcasted_iota(jnp.int32, (_B, _B), 0) * _B
                + jax.lax.broadcasted_iota(jnp.int32, (_B, _B), 1))
        row8 = jax.lax.broadcasted_iota(jnp.int32, (_B, _B), 0)
        col8 = jax.lax.broadcasted_iota(jnp.int32, (_B, _B), 1)

        a_acc = jnp.zeros((1, _B), jnp.float32)
        lp_acc = jnp.zeros((1, _B), jnp.float32)
        en_acc = jnp.zeros((1, _B), jnp.float32)
        tk_acc = jnp.zeros((1, _B), jnp.int32)
        onehot = jnp.zeros((_B, _B), jnp.float32)
        energy_b = jnp.broadcast_to(energy, (_B, _B))

        aw = action
        for j2 in range(_B):
            a = jnp.min(aw)
            sel = jnp.min(jnp.where(aw <= a, cidx, _IBIG))
            hit = cidx == sel
            a_acc = jnp.where(lane8 == j2, a, a_acc)
            lp_acc = jnp.where(lane8 == j2,
                               jnp.sum(jnp.where(hit, cand_logp, 0.0)), lp_acc)
            en_acc = jnp.where(lane8 == j2,
                               jnp.sum(jnp.where(hit, energy_b, 0.0)), en_acc)
            tk_acc = jnp.where(lane8 == j2, jnp.sum(jnp.where(hit, topi, 0)),
                               tk_acc)
            beam = sel // _B
            onehot = onehot + jnp.where((row8 == j2) & (col8 == beam),
                                        1.0, 0.0)
            aw = jnp.where(hit, jnp.inf, aw)

        act_ref[...] = a_acc
        logp_ref[...] = lp_acc
        eng_ref[...] = en_acc
        tok_ref[...] = tk_acc
        hid_out_ref[...] = jax.lax.dot(onehot, hid_ref[...],
                                       preferred_element_type=jnp.float32)


@functools.partial(jax.jit, static_argnames=("interpret",))
def _impl(next_token_logits, next_hidden_states, energies, interpret=False):
    ids, stats = pl.pallas_call(
        _scan_body,
        grid=(_NC,),
        in_specs=[pl.BlockSpec((_B, _C), lambda i: (0, i))],
        out_specs=[
            pl.BlockSpec((_B, _B), lambda i: (0, 0)),
            pl.BlockSpec((_B, 128), lambda i: (0, 0)),
        ],
        out_shape=[
            jax.ShapeDtypeStruct((_B, _B), jnp.int32),
            jax.ShapeDtypeStruct((_B, 128), jnp.float32),
        ],
        scratch_shapes=[
            pltpu.VMEM((_B, 128), jnp.float32),
            pltpu.VMEM((_B, 128), jnp.float32),
            pltpu.VMEM((_B, _NC * _GPC), jnp.float32),
        ],
        interpret=interpret,
    )(next_token_logits)

    grid_spec = pltpu.PrefetchScalarGridSpec(
        num_scalar_prefetch=1,
        grid=(_B, _B),
        in_specs=[
            pl.BlockSpec((_B, _G), lambda b, j, ids: (0, ids[b * _B + j])),
            pl.BlockSpec((_B, 128), lambda b, j, ids: (0, 0)),
            pl.BlockSpec((_B, 1), lambda b, j, ids: (0, 0)),
            pl.BlockSpec((_B, 4096), lambda b, j, ids: (0, 0)),
        ],
        out_specs=[
            pl.BlockSpec((1, _B), lambda b, j, ids: (0, 0)),
            pl.BlockSpec((1, _B), lambda b, j, ids: (0, 0)),
            pl.BlockSpec((1, _B), lambda b, j, ids: (0, 0)),
            pl.BlockSpec((1, _B), lambda b, j, ids: (0, 0)),
            pl.BlockSpec((_B, 4096), lambda b, j, ids: (0, 0)),
        ],
        scratch_shapes=[pltpu.VMEM((_B * _B, _G), jnp.float32)],
    )
    act, logp, eng, tok, hid = pl.pallas_call(
        _merge_body,
        grid_spec=grid_spec,
        out_shape=[
            jax.ShapeDtypeStruct((1, _B), jnp.float32),
            jax.ShapeDtypeStruct((1, _B), jnp.float32),
            jax.ShapeDtypeStruct((1, _B), jnp.float32),
            jax.ShapeDtypeStruct((1, _B), jnp.int32),
            jax.ShapeDtypeStruct((_B, 4096), jnp.float32),
        ],
        interpret=interpret,
    )(ids.reshape(_B * _B), next_token_logits, stats,
      energies.reshape(_B, 1), next_hidden_states)

    return (act.reshape(_B), logp.reshape(_B), eng.reshape(_B),
            tok.reshape(_B), hid)


def kernel(next_token_logits, next_hidden_states, energies):
    return _impl(next_token_logits, next_hidden_states, energies)


# speedup vs baseline: 4.4493x; 2.1138x over previous
import functools

import jax
import jax.numpy as jnp
from jax.experimental import pallas as pl
from jax.experimental.pallas import tpu as pltpu

_B = 8
_V = 1000000
_C = 32768
_NC = 31
_G = 1024
_GPC = _C // _G
_NG = _NC * _GPC
_NEG = -jnp.inf
_IBIG = 2**30


def _treemax(xs):
    xs = list(xs)
    while len(xs) > 1:
        nxt = [jnp.maximum(xs[k], xs[k + 1]) for k in range(0, len(xs) - 1, 2)]
        if len(xs) % 2:
            nxt.append(xs[-1])
        xs = nxt
    return xs[0]


def _scan_chunk(x, gm_ref, m_scr, s_scr):
    groups = [[x[:, (j * 8 + t) * 128:(j * 8 + t + 1) * 128] for t in range(8)]
              for j in range(_GPC)]
    g8 = [_treemax(g) for g in groups]
    gm_cols = [jnp.max(g, axis=1, keepdims=True) for g in g8]
    gm_ref[0] = jnp.concatenate(gm_cols, axis=1)

    gfold = _treemax(g8)
    m_c = jnp.max(gfold, axis=1, keepdims=True)
    m_old = m_scr[...]
    m_new = jnp.maximum(m_old, m_c)
    mb = m_new[:, 0:1]

    accs = [jnp.zeros((_B, 128), jnp.float32) for _ in range(4)]
    for j in range(_GPC):
        for t in range(8):
            k = j * 8 + t
            accs[k % 4] = accs[k % 4] + jnp.exp(groups[j][t] - mb)
    acc = (accs[0] + accs[1]) + (accs[2] + accs[3])

    s_scr[...] = s_scr[...] * jnp.exp(m_old - m_new) + acc
    m_scr[...] = m_new


def _scan_body(x_ref, gm_ref, stats_ref, m_scr, s_scr):
    i = pl.program_id(0)

    @pl.when(i == 0)
    def _init():
        m_scr[...] = jnp.full((_B, 128), _NEG, jnp.float32)
        s_scr[...] = jnp.zeros((_B, 128), jnp.float32)

    @pl.when(i < _NC - 1)
    def _full():
        _scan_chunk(x_ref[...], gm_ref, m_scr, s_scr)

    @pl.when(i == _NC - 1)
    def _edge():
        tail = _V - (_NC - 1) * _C
        liota = jax.lax.broadcasted_iota(jnp.int32, (_B, _C), 1)
        x = jnp.where(liota < tail, x_ref[...], _NEG)
        _scan_chunk(x, gm_ref, m_scr, s_scr)
        l128 = jax.lax.broadcasted_iota(jnp.int32, (_B, 128), 1)
        s_b = jnp.sum(s_scr[...], axis=1, keepdims=True)
        stats_ref[...] = jnp.where(l128 == 0, m_scr[...],
                                   jnp.where(l128 == 1, s_b, 0.0))


def _select_body(gm_ref, ids_ref):
    gmv = gm_ref[...]
    giota = jax.lax.broadcasted_iota(jnp.int32, (_B, _NG), 1)
    lane8 = jax.lax.broadcasted_iota(jnp.int32, (1, _B), 1)
    ids = jnp.zeros((_B, _B), jnp.int32)
    for j in range(_B):
        v = jnp.max(gmv, axis=1, keepdims=True)
        sel = jnp.min(jnp.where(gmv >= v, giota, _IBIG),
                      axis=1, keepdims=True)
        ids = jnp.where(lane8 == j, sel, ids)
        gmv = jnp.where(giota == sel, _NEG, gmv)
    ids_ref[...] = ids


def _merge_body(ids_ref, *refs):
    xrefs = refs[:_B * _B]
    stats_ref, energy_ref, hid_ref = refs[_B * _B:_B * _B + 3]
    act_ref, logp_ref, eng_ref, tok_ref, hid_out_ref = refs[_B * _B + 3:]

    li = jax.lax.broadcasted_iota(jnp.int32, (1, _G), 1)
    rows = []
    lidx_rows = []
    for b in range(_B):
        rows.append(jnp.concatenate(
            [xrefs[b * _B + j][b:b + 1, :] for j in range(_B)], axis=1))
        lidx_rows.append(jnp.concatenate(
            [ids_ref[b * _B + j] * _G + li for j in range(_B)], axis=1))
    cand = jnp.concatenate(rows, axis=0)
    lidx = jnp.concatenate(lidx_rows, axis=0)
    cand = jnp.where(lidx < _V, cand, _NEG)

    lane8 = jax.lax.broadcasted_iota(jnp.int32, (1, _B), 1)
    topv = jnp.zeros((_B, _B), jnp.float32)
    topi = jnp.zeros((_B, _B), jnp.int32)
    for j in range(_B):
        v = jnp.max(cand, axis=1, keepdims=True)
        sel = jnp.min(jnp.where(cand >= v, lidx, _IBIG),
                      axis=1, keepdims=True)
        topv = jnp.where(lane8 == j, v, topv)
        topi = jnp.where(lane8 == j, sel, topi)
        cand = jnp.where(lidx == sel, _NEG, cand)

    stats = stats_ref[...]
    m = stats[:, 0:1]
    s = stats[:, 1:2]
    energy = energy_ref[...]
    cand_logp = (topv - m) - jnp.log(s)
    action = energy - cand_logp

    cidx = (jax.lax.broadcasted_iota(jnp.int32, (_B, _B), 0) * _B
            + jax.lax.broadcasted_iota(jnp.int32, (_B, _B), 1))
    row8 = jax.lax.broadcasted_iota(jnp.int32, (_B, _B), 0)
    col8 = jax.lax.broadcasted_iota(jnp.int32, (_B, _B), 1)

    a_acc = jnp.zeros((1, _B), jnp.float32)
    lp_acc = jnp.zeros((1, _B), jnp.float32)
    en_acc = jnp.zeros((1, _B), jnp.float32)
    tk_acc = jnp.zeros((1, _B), jnp.int32)
    onehot = jnp.zeros((_B, _B), jnp.float32)
    energy_b = jnp.broadcast_to(energy, (_B, _B))

    aw = action
    for j in range(_B):
        a = jnp.min(aw)
        sel = jnp.min(jnp.where(aw <= a, cidx, _IBIG))
        hit = cidx == sel
        a_acc = jnp.where(lane8 == j, a, a_acc)
        lp_acc = jnp.where(lane8 == j,
                           jnp.sum(jnp.where(hit, cand_logp, 0.0)), lp_acc)
        en_acc = jnp.where(lane8 == j,
                           jnp.sum(jnp.where(hit, energy_b, 0.0)), en_acc)
        tk_acc = jnp.where(lane8 == j, jnp.sum(jnp.where(hit, topi, 0)),
                           tk_acc)
        beam = sel // _B
        onehot = onehot + jnp.where((row8 == j) & (col8 == beam), 1.0, 0.0)
        aw = jnp.where(hit, jnp.inf, aw)

    act_ref[...] = a_acc
    logp_ref[...] = lp_acc
    eng_ref[...] = en_acc
    tok_ref[...] = tk_acc
    hid_out_ref[...] = jax.lax.dot(onehot, hid_ref[...],
                                   preferred_element_type=jnp.float32)


@functools.partial(jax.jit, static_argnames=("interpret",))
def _impl(next_token_logits, next_hidden_states, energies, interpret=False):
    gm, stats = pl.pallas_call(
        _scan_body,
        grid=(_NC,),
        in_specs=[pl.BlockSpec((_B, _C), lambda i: (0, i))],
        out_specs=[
            pl.BlockSpec((1, _B, _GPC), lambda i: (i, 0, 0)),
            pl.BlockSpec((_B, 128), lambda i: (0, 0)),
        ],
        out_shape=[
            jax.ShapeDtypeStruct((_NC, _B, _GPC), jnp.float32),
            jax.ShapeDtypeStruct((_B, 128), jnp.float32),
        ],
        scratch_shapes=[
            pltpu.VMEM((_B, 128), jnp.float32),
            pltpu.VMEM((_B, 128), jnp.float32),
        ],
        interpret=interpret,
    )(next_token_logits)

    gm2 = jnp.transpose(gm, (1, 0, 2)).reshape(_B, _NG)
    ids = pl.pallas_call(
        _select_body,
        out_shape=jax.ShapeDtypeStruct((_B, _B), jnp.int32),
        interpret=interpret,
    )(gm2)

    def _xmap(r):
        return lambda i, ids_pf: (0, ids_pf[r])

    grid_spec = pltpu.PrefetchScalarGridSpec(
        num_scalar_prefetch=1,
        grid=(1,),
        in_specs=(
            [pl.BlockSpec((_B, _G), _xmap(r)) for r in range(_B * _B)]
            + [
                pl.BlockSpec((_B, 128), lambda i, ids_pf: (0, 0)),
                pl.BlockSpec((_B, 1), lambda i, ids_pf: (0, 0)),
                pl.BlockSpec((_B, 4096), lambda i, ids_pf: (0, 0)),
            ]),
        out_specs=[
            pl.BlockSpec((1, _B), lambda i, ids_pf: (0, 0)),
            pl.BlockSpec((1, _B), lambda i, ids_pf: (0, 0)),
            pl.BlockSpec((1, _B), lambda i, ids_pf: (0, 0)),
            pl.BlockSpec((1, _B), lambda i, ids_pf: (0, 0)),
            pl.BlockSpec((_B, 4096), lambda i, ids_pf: (0, 0)),
        ],
    )
    act, logp, eng, tok, hid = pl.pallas_call(
        _merge_body,
        grid_spec=grid_spec,
        out_shape=[
            jax.ShapeDtypeStruct((1, _B), jnp.float32),
            jax.ShapeDtypeStruct((1, _B), jnp.float32),
            jax.ShapeDtypeStruct((1, _B), jnp.float32),
            jax.ShapeDtypeStruct((1, _B), jnp.int32),
            jax.ShapeDtypeStruct((_B, 4096), jnp.float32),
        ],
        interpret=interpret,
    )(ids.reshape(_B * _B), *([next_token_logits] * (_B * _B)),
      stats, energies.reshape(_B, 1), next_hidden_states)

    return (act.reshape(_B), logp.reshape(_B), eng.reshape(_B),
            tok.reshape(_B), hid)


def kernel(next_token_logits, next_hidden_states, energies):
    return _impl(next_token_logits, next_hidden_states, energies)


# speedup vs baseline: 5.5109x; 1.2386x over previous
import functools

import jax
import jax.numpy as jnp
from jax.experimental import pallas as pl
from jax.experimental.pallas import tpu as pltpu

_B = 8
_V = 1000000
_C = 65536
_NC = 16
_G = 1024
_GPC = _C // _G
_NG = _NC * _GPC
_NEG = -jnp.inf
_IBIG = 2**30


def _treemax(xs):
    xs = list(xs)
    while len(xs) > 1:
        nxt = [jnp.maximum(xs[k], xs[k + 1]) for k in range(0, len(xs) - 1, 2)]
        if len(xs) % 2:
            nxt.append(xs[-1])
        xs = nxt
    return xs[0]


def _scan_chunk(x, gm_ref, m_scr, s_scr):
    groups = [[x[:, (j * 8 + t) * 128:(j * 8 + t + 1) * 128] for t in range(8)]
              for j in range(_GPC)]
    g8 = [_treemax(g) for g in groups]
    gm_cols = [jnp.max(g, axis=1, keepdims=True) for g in g8]
    gm_ref[0] = jnp.concatenate(gm_cols, axis=1)

    gfold = _treemax(g8)
    m_c = jnp.max(gfold, axis=1, keepdims=True)
    m_old = m_scr[...]
    m_new = jnp.maximum(m_old, m_c)
    mb = m_new[:, 0:1]

    accs = [jnp.zeros((_B, 128), jnp.float32) for _ in range(4)]
    for j in range(_GPC):
        for t in range(8):
            k = j * 8 + t
            accs[k % 4] = accs[k % 4] + jnp.exp(groups[j][t] - mb)
    acc = (accs[0] + accs[1]) + (accs[2] + accs[3])

    s_scr[...] = s_scr[...] * jnp.exp(m_old - m_new) + acc
    m_scr[...] = m_new


def _scan_body(x_ref, gm_ref, stats_ref, m_scr, s_scr):
    i = pl.program_id(0)

    @pl.when(i == 0)
    def _init():
        m_scr[...] = jnp.full((_B, 128), _NEG, jnp.float32)
        s_scr[...] = jnp.zeros((_B, 128), jnp.float32)

    @pl.when(i < _NC - 1)
    def _full():
        _scan_chunk(x_ref[...], gm_ref, m_scr, s_scr)

    @pl.when(i == _NC - 1)
    def _edge():
        tail = _V - (_NC - 1) * _C
        liota = jax.lax.broadcasted_iota(jnp.int32, (_B, _C), 1)
        x = jnp.where(liota < tail, x_ref[...], _NEG)
        _scan_chunk(x, gm_ref, m_scr, s_scr)
        l128 = jax.lax.broadcasted_iota(jnp.int32, (_B, 128), 1)
        s_b = jnp.sum(s_scr[...], axis=1, keepdims=True)
        stats_ref[...] = jnp.where(l128 == 0, m_scr[...],
                                   jnp.where(l128 == 1, s_b, 0.0))


def _select_body(gm_ref, ids_ref):
    gmv = gm_ref[...]
    giota = (jax.lax.broadcasted_iota(jnp.int32, (_NC, _B, _GPC), 0) * _GPC
             + jax.lax.broadcasted_iota(jnp.int32, (_NC, _B, _GPC), 2))
    lane8 = jax.lax.broadcasted_iota(jnp.int32, (1, _B), 1)
    ids = jnp.zeros((_B, _B), jnp.int32)
    for j in range(_B):
        v = jnp.max(jnp.max(gmv, axis=0), axis=1,
                    keepdims=True)
        u = jnp.min(jnp.where(gmv >= v.reshape(1, _B, 1), giota, _IBIG),
                    axis=0)
        sel = jnp.min(u, axis=1, keepdims=True)
        ids = jnp.where(lane8 == j, sel, ids)
        gmv = jnp.where(giota == sel.reshape(1, _B, 1), _NEG, gmv)
    ids_ref[...] = ids


def _merge_body(ids_ref, *refs):
    xrefs = refs[:_B * _B]
    stats_ref, energy_ref, hid_ref = refs[_B * _B:_B * _B + 3]
    act_ref, logp_ref, eng_ref, tok_ref, hid_out_ref = refs[_B * _B + 3:]

    li = jax.lax.broadcasted_iota(jnp.int32, (1, _G), 1)
    rows = []
    lidx_rows = []
    for b in range(_B):
        rows.append(jnp.concatenate(
            [xrefs[b * _B + j][b:b + 1, :] for j in range(_B)], axis=1))
        lidx_rows.append(jnp.concatenate(
            [ids_ref[b * _B + j] * _G + li for j in range(_B)], axis=1))
    cand = jnp.concatenate(rows, axis=0)
    lidx = jnp.concatenate(lidx_rows, axis=0)
    cand = jnp.where(lidx < _V, cand, _NEG)

    lane8 = jax.lax.broadcasted_iota(jnp.int32, (1, _B), 1)
    topv = jnp.zeros((_B, _B), jnp.float32)
    topi = jnp.zeros((_B, _B), jnp.int32)
    for j in range(_B):
        v = jnp.max(cand, axis=1, keepdims=True)
        sel = jnp.min(jnp.where(cand >= v, lidx, _IBIG),
                      axis=1, keepdims=True)
        topv = jnp.where(lane8 == j, v, topv)
        topi = jnp.where(lane8 == j, sel, topi)
        cand = jnp.where(lidx == sel, _NEG, cand)

    stats = stats_ref[...]
    m = stats[:, 0:1]
    s = stats[:, 1:2]
    energy = energy_ref[...]
    cand_logp = (topv - m) - jnp.log(s)
    action = energy - cand_logp

    cidx = (jax.lax.broadcasted_iota(jnp.int32, (_B, _B), 0) * _B
            + jax.lax.broadcasted_iota(jnp.int32, (_B, _B), 1))
    row8 = jax.lax.broadcasted_iota(jnp.int32, (_B, _B), 0)
    col8 = jax.lax.broadcasted_iota(jnp.int32, (_B, _B), 1)

    a_acc = jnp.zeros((1, _B), jnp.float32)
    lp_acc = jnp.zeros((1, _B), jnp.float32)
    en_acc = jnp.zeros((1, _B), jnp.float32)
    tk_acc = jnp.zeros((1, _B), jnp.int32)
    onehot = jnp.zeros((_B, _B), jnp.float32)
    energy_b = jnp.broadcast_to(energy, (_B, _B))

    aw = action
    for j in range(_B):
        a = jnp.min(aw)
        sel = jnp.min(jnp.where(aw <= a, cidx, _IBIG))
        hit = cidx == sel
        a_acc = jnp.where(lane8 == j, a, a_acc)
        lp_acc = jnp.where(lane8 == j,
                           jnp.sum(jnp.where(hit, cand_logp, 0.0)), lp_acc)
        en_acc = jnp.where(lane8 == j,
                           jnp.sum(jnp.where(hit, energy_b, 0.0)), en_acc)
        tk_acc = jnp.where(lane8 == j, jnp.sum(jnp.where(hit, topi, 0)),
                           tk_acc)
        beam = sel // _B
        onehot = onehot + jnp.where((row8 == j) & (col8 == beam), 1.0, 0.0)
        aw = jnp.where(hit, jnp.inf, aw)

    act_ref[...] = a_acc
    logp_ref[...] = lp_acc
    eng_ref[...] = en_acc
    tok_ref[...] = tk_acc
    hid_out_ref[...] = jax.lax.dot(onehot, hid_ref[...],
                                   preferred_element_type=jnp.float32)


@functools.partial(jax.jit, static_argnames=("interpret",))
def _impl(next_token_logits, next_hidden_states, energies, interpret=False):
    gm, stats = pl.pallas_call(
        _scan_body,
        grid=(_NC,),
        in_specs=[pl.BlockSpec((_B, _C), lambda i: (0, i))],
        out_specs=[
            pl.BlockSpec((1, _B, _GPC), lambda i: (i, 0, 0)),
            pl.BlockSpec((_B, 128), lambda i: (0, 0)),
        ],
        out_shape=[
            jax.ShapeDtypeStruct((_NC, _B, _GPC), jnp.float32),
            jax.ShapeDtypeStruct((_B, 128), jnp.float32),
        ],
        scratch_shapes=[
            pltpu.VMEM((_B, 128), jnp.float32),
            pltpu.VMEM((_B, 128), jnp.float32),
        ],
        interpret=interpret,
    )(next_token_logits)

    ids = pl.pallas_call(
        _select_body,
        out_shape=jax.ShapeDtypeStruct((_B, _B), jnp.int32),
        interpret=interpret,
    )(gm)

    def _xmap(r):
        return lambda i, ids_pf: (0, ids_pf[r])

    grid_spec = pltpu.PrefetchScalarGridSpec(
        num_scalar_prefetch=1,
        grid=(1,),
        in_specs=(
            [pl.BlockSpec((_B, _G), _xmap(r)) for r in range(_B * _B)]
            + [
                pl.BlockSpec((_B, 128), lambda i, ids_pf: (0, 0)),
                pl.BlockSpec((_B, 1), lambda i, ids_pf: (0, 0)),
                pl.BlockSpec((_B, 4096), lambda i, ids_pf: (0, 0)),
            ]),
        out_specs=[
            pl.BlockSpec((1, _B), lambda i, ids_pf: (0, 0)),
            pl.BlockSpec((1, _B), lambda i, ids_pf: (0, 0)),
            pl.BlockSpec((1, _B), lambda i, ids_pf: (0, 0)),
            pl.BlockSpec((1, _B), lambda i, ids_pf: (0, 0)),
            pl.BlockSpec((_B, 4096), lambda i, ids_pf: (0, 0)),
        ],
    )
    act, logp, eng, tok, hid = pl.pallas_call(
        _merge_body,
        grid_spec=grid_spec,
        out_shape=[
            jax.ShapeDtypeStruct((1, _B), jnp.float32),
            jax.ShapeDtypeStruct((1, _B), jnp.float32),
            jax.ShapeDtypeStruct((1, _B), jnp.float32),
            jax.ShapeDtypeStruct((1, _B), jnp.int32),
            jax.ShapeDtypeStruct((_B, 4096), jnp.float32),
        ],
        interpret=interpret,
    )(ids.reshape(_B * _B), *([next_token_logits] * (_B * _B)),
      stats, energies.reshape(_B, 1), next_hidden_states)

    return (act.reshape(_B), logp.reshape(_B), eng.reshape(_B),
            tok.reshape(_B), hid)


def kernel(next_token_logits, next_hidden_states, energies):
    return _impl(next_token_logits, next_hidden_states, energies)
